# baseline (speedup 1.0000x reference)
import jax
import jax.numpy as jnp
from jax import lax
from jax.experimental import pallas as pl
from jax.experimental.pallas import tpu as pltpu
from jax.experimental.pallas import tpu_sc as plsc

_N = 10000
_E = 320000
_D = 128

_NC = 2
_NS = 16
_NW = _NC * _NS
_L = 16

_C = 80
_P = 4
_E2 = 2 * _E
_CHUNKS = -(-_E2 // (_NW * _C * _P)) * _P
_CA = 192
_CB = 2 * _CHUNKS - _CA
_PER_TILE = _CHUNKS * _C
_EP = 2 * _PER_TILE * _NS

_NPAD = 10240
_RPS = _NPAD // _NS

_BLK = 1000


def _dense_body(f_ref, wl_ref, wi_ref, wl1_ref, wi1_ref, y_ref):
    x = f_ref[...]
    x2 = x * x
    dn = (((1,), (1,)), ((), ()))
    y_ref[0] = (lax.dot_general(x, wl_ref[...], dn, preferred_element_type=jnp.float32)
                + lax.dot_general(x2, wi_ref[...], dn, preferred_element_type=jnp.float32))
    y_ref[1] = (lax.dot_general(x, wl1_ref[...], dn, preferred_element_type=jnp.float32)
                + lax.dot_general(x2, wi1_ref[...], dn, preferred_element_type=jnp.float32))


def _dense(features, wl, wi, wl1, wi1):
    w_spec = pl.BlockSpec((_D, _D), lambda i: (0, 0))
    return pl.pallas_call(
        _dense_body,
        grid=(_N // _BLK,),
        in_specs=[pl.BlockSpec((_BLK, _D), lambda i: (i, 0))] + [w_spec] * 4,
        out_specs=pl.BlockSpec((2, _BLK, _D), lambda i: (0, i, 0)),
        out_shape=jax.ShapeDtypeStruct((2, _N, _D), jnp.float32),
    )(features, wl, wi, wl1, wi1)


def _spmm_body(y2, colsf, rowsf, valsf, out, cbufs, rbufs, vbufs, gbufs, acc, sis, sgs, sss):
    c = lax.axis_index("c")
    s = lax.axis_index("s")

    def zrow(i, carry):
        for k in range(_D // _L):
            gbufs[0][i, pl.ds(k * _L, _L)] = jnp.zeros((_L,), jnp.float32)
        return carry

    lax.fori_loop(0, _C, zrow, 0)
    for j in range(_RPS // _C):
        pltpu.sync_copy(gbufs[0], acc.at[pl.ds(s * _RPS + j * _C, _C)])
    plsc.subcore_barrier()

    cbase = s * (_CA + _CB) + c * _CA
    nchunks = jnp.where(c == 0, _CA, _CB)

    def start_idx(g, b):
        e0 = (cbase + g) * _C
        pltpu.async_copy(colsf.at[pl.ds(e0, _C)], cbufs[b], sis[b])
        pltpu.async_copy(rowsf.at[pl.ds(e0, _C)], rbufs[b], sis[b])
        pltpu.async_copy(valsf.at[pl.ds(e0, _C)], vbufs[b], sis[b])

    def wait_idx(g, b):
        e0 = (cbase + g) * _C
        pltpu.make_async_copy(colsf.at[pl.ds(e0, _C)], cbufs[b], sis[b]).wait()
        pltpu.make_async_copy(rowsf.at[pl.ds(e0, _C)], rbufs[b], sis[b]).wait()
        pltpu.make_async_copy(valsf.at[pl.ds(e0, _C)], vbufs[b], sis[b]).wait()

    def start_gather(b):
        pltpu.async_copy(y2.at[cbufs[b]], gbufs[b], sgs[b])

    def wait_gather(b):
        pltpu.make_async_copy(y2.at[cbufs[b]], gbufs[b], sgs[b]).wait()

    def start_scatter(b):
        pltpu.async_copy(gbufs[b], acc.at[rbufs[b]], sss[b], add=True)

    def wait_scatter(b):
        pltpu.make_async_copy(gbufs[b], acc.at[rbufs[b]], sss[b]).wait()

    start_idx(0, 0)
    start_idx(1, 1)
    wait_idx(0, 0)
    start_gather(0)

    def outer(g4, carry):
        for u in range(_P):
            b = u
            g = g4 * _P + u
            wait_gather(b)
            b2 = (u + 2) % _P

            @pl.when(g >= 2)
            def _():
                wait_scatter(b2)

            start_idx(g + 2, b2)
            b1 = (u + 1) % _P
            wait_idx(g + 1, b1)
            start_gather(b1)

            gb = gbufs[b]
            vb = vbufs[b]

            def grp(t, carry2):
                vvec = vb[pl.ds(t * _L, _L)]
                for j in range(_L):
                    e = t * _L + j
                    v = jnp.full((_L,), vvec[j], jnp.float32)
                    for k in range(_D // _L):
                        gb[e, pl.ds(k * _L, _L)] = gb[e, pl.ds(k * _L, _L)] * v
                return carry2

            lax.fori_loop(0, _C // _L, grp, 0)
            start_scatter(b)
        return carry

    lax.fori_loop(0, nchunks // _P, outer, 0)
    wait_gather(0)
    wait_idx(nchunks + 1, 1)
    for b in (_P - 2, _P - 1):
        wait_scatter(b)
    plsc.subcore_barrier()

    for j in range(_RPS // _C):
        r0 = s * _RPS + j * _C
        pltpu.sync_copy(acc.at[pl.ds(r0, _C)], gbufs[0])
        pltpu.sync_copy(gbufs[0], out.at[c, pl.ds(r0, _C)])


def _spmm(y2, colsf, rowsf, valsf):
    mesh = plsc.VectorSubcoreMesh(core_axis_name="c", subcore_axis_name="s")

    def body(y2_ref, colsf_ref, rowsf_ref, valsf_ref, out_ref, *scratch):
        cbufs = scratch[0:_P]
        rbufs = scratch[_P:2 * _P]
        vbufs = scratch[2 * _P:3 * _P]
        gbufs = scratch[3 * _P:4 * _P]
        acc = scratch[4 * _P]
        sems = scratch[4 * _P + 1:]
        _spmm_body(y2_ref, colsf_ref, rowsf_ref, valsf_ref, out_ref,
                   cbufs, rbufs, vbufs, gbufs, acc,
                   sems[0:_P], sems[_P:2 * _P], sems[2 * _P:3 * _P])

    return pl.kernel(
        body,
        out_type=jax.ShapeDtypeStruct((_NC, _NPAD, _D), jnp.float32),
        mesh=mesh,
        scratch_types=(
            [pltpu.VMEM((_C,), jnp.int32)] * _P
            + [pltpu.VMEM((_C,), jnp.int32)] * _P
            + [pltpu.VMEM((_C,), jnp.float32)] * _P
            + [pltpu.VMEM((_C, _D), jnp.float32)] * _P
            + [pltpu.VMEM_SHARED((_NPAD, _D), jnp.float32)]
            + [pltpu.SemaphoreType.DMA] * (3 * _P)
        ),
    )(y2, colsf, rowsf, valsf)


def _combine_body(p_ref, b_ref, o_ref):
    o_ref[...] = p_ref[0] + p_ref[1] + b_ref[...]


def _combine(partials, bias):
    return pl.pallas_call(
        _combine_body,
        grid=(_N // _BLK,),
        in_specs=[pl.BlockSpec((2, _BLK, _D), lambda i: (0, i, 0)),
                  pl.BlockSpec((1, _D), lambda i: (0, 0))],
        out_specs=pl.BlockSpec((_BLK, _D), lambda i: (i, 0)),
        out_shape=jax.ShapeDtypeStruct((_N, _D), jnp.float32),
    )(partials, bias)


def kernel(features, laplacian_indices, laplacian_values, selfloop_indices,
           selfloop_values, ui_indices, ui_values,
           W_lin, b_lin, W_lin1, b_lin1, W_iat, b_iat, W_iat1, b_iat1):
    y = _dense(features, W_lin, W_iat, W_lin1, W_iat1)
    y2 = y.reshape(2 * _N, _D)

    pad = _EP + 2 * _C - _E2
    cols = jnp.concatenate([
        laplacian_indices[1], ui_indices[1] + _N,
        jnp.zeros((pad,), jnp.int32)])
    rows = jnp.concatenate([
        laplacian_indices[0], ui_indices[0],
        jnp.zeros((pad,), jnp.int32)])
    vals = jnp.concatenate([
        laplacian_values, ui_values, jnp.zeros((pad,), jnp.float32)])
    partials = _spmm(y2, cols, rows, vals)

    bias = (b_lin + b_iat + b_lin1 + b_iat1).reshape(1, _D)
    return _combine(partials, bias)


# speedup vs baseline: 1.2126x; 1.2126x over previous
import jax
import jax.numpy as jnp
from jax import lax
from jax.experimental import pallas as pl
from jax.experimental.pallas import tpu as pltpu
from jax.experimental.pallas import tpu_sc as plsc

_N = 10000
_E = 320000
_D = 128

_NC = 2
_NS = 16
_NW = _NC * _NS
_L = 16

_C = 80
_P = 4
_E2 = 2 * _E
_CHUNKS = -(-_E2 // (_NW * _C * _P)) * _P
_CA = 312
_CB = 2 * _CHUNKS - _CA
_PER_TILE = _CHUNKS * _C
_EP = 2 * _PER_TILE * _NS

_NPAD = 10240
_RPS = _NPAD // _NS

_BLK = 1000


def _dense_body(f_ref, wl_ref, wi_ref, wl1_ref, wi1_ref, y_ref):
    x = f_ref[...]
    x2 = x * x
    dn = (((1,), (1,)), ((), ()))
    y_ref[0] = (lax.dot_general(x, wl_ref[...], dn, preferred_element_type=jnp.float32)
                + lax.dot_general(x2, wi_ref[...], dn, preferred_element_type=jnp.float32))
    y_ref[1] = (lax.dot_general(x, wl1_ref[...], dn, preferred_element_type=jnp.float32)
                + lax.dot_general(x2, wi1_ref[...], dn, preferred_element_type=jnp.float32))


def _dense(features, wl, wi, wl1, wi1):
    w_spec = pl.BlockSpec((_D, _D), lambda i: (0, 0))
    return pl.pallas_call(
        _dense_body,
        grid=(_N // _BLK,),
        in_specs=[pl.BlockSpec((_BLK, _D), lambda i: (i, 0))] + [w_spec] * 4,
        out_specs=pl.BlockSpec((2, _BLK, _D), lambda i: (0, i, 0)),
        out_shape=jax.ShapeDtypeStruct((2, _N, _D), jnp.float32),
    )(features, wl, wi, wl1, wi1)


def _spmm_body(y2, colsf, rowsf, valsf, out, cbufs, rbufs, vbufs, gbufs, acc, sis, sgs, sss):
    c = lax.axis_index("c")
    s = lax.axis_index("s")

    def zrow(i, carry):
        for k in range(_D // _L):
            gbufs[0][i, pl.ds(k * _L, _L)] = jnp.zeros((_L,), jnp.float32)
        return carry

    lax.fori_loop(0, _C, zrow, 0)
    for j in range(_RPS // _C):
        pltpu.sync_copy(gbufs[0], acc.at[pl.ds(s * _RPS + j * _C, _C)])
    plsc.subcore_barrier()

    cbase = s * (_CA + _CB) + c * _CA
    nchunks = jnp.where(c == 0, _CA, _CB)

    def start_idx(g, b):
        e0 = (cbase + g) * _C
        pltpu.async_copy(colsf.at[pl.ds(e0, _C)], cbufs[b], sis[b])
        pltpu.async_copy(rowsf.at[pl.ds(e0, _C)], rbufs[b], sis[b])
        pltpu.async_copy(valsf.at[pl.ds(e0, _C)], vbufs[b], sis[b])

    def wait_idx(g, b):
        e0 = (cbase + g) * _C
        pltpu.make_async_copy(colsf.at[pl.ds(e0, _C)], cbufs[b], sis[b]).wait()
        pltpu.make_async_copy(rowsf.at[pl.ds(e0, _C)], rbufs[b], sis[b]).wait()
        pltpu.make_async_copy(valsf.at[pl.ds(e0, _C)], vbufs[b], sis[b]).wait()

    def start_gather(b):
        pltpu.async_copy(y2.at[cbufs[b]], gbufs[b], sgs[b])

    def wait_gather(b):
        pltpu.make_async_copy(y2.at[cbufs[b]], gbufs[b], sgs[b]).wait()

    def start_scatter(b):
        pltpu.async_copy(gbufs[b], acc.at[rbufs[b]], sss[b], add=True)

    def wait_scatter(b):
        pltpu.make_async_copy(gbufs[b], acc.at[rbufs[b]], sss[b]).wait()

    start_idx(0, 0)
    start_idx(1, 1)
    wait_idx(0, 0)
    start_gather(0)

    def outer(g4, carry):
        for u in range(_P):
            b = u
            g = g4 * _P + u
            wait_gather(b)
            b2 = (u + 2) % _P

            @pl.when(g >= 2)
            def _():
                wait_scatter(b2)

            start_idx(g + 2, b2)
            b1 = (u + 1) % _P
            wait_idx(g + 1, b1)
            start_gather(b1)

            gb = gbufs[b]
            vb = vbufs[b]

            def grp(t, carry2):
                vvec = vb[pl.ds(t * _L, _L)]
                for j in range(_L):
                    e = t * _L + j
                    v = jnp.full((_L,), vvec[j], jnp.float32)
                    for k in range(_D // _L):
                        gb[e, pl.ds(k * _L, _L)] = gb[e, pl.ds(k * _L, _L)] * v
                return carry2

            lax.fori_loop(0, _C // _L, grp, 0)
            start_scatter(b)
        return carry

    lax.fori_loop(0, nchunks // _P, outer, 0)
    wait_gather(0)
    wait_idx(nchunks + 1, 1)
    for b in (_P - 2, _P - 1):
        wait_scatter(b)
    plsc.subcore_barrier()

    for j in range(_RPS // _C):
        r0 = s * _RPS + j * _C
        pltpu.sync_copy(acc.at[pl.ds(r0, _C)], gbufs[0])
        pltpu.sync_copy(gbufs[0], out.at[c, pl.ds(r0, _C)])


def _spmm(y2, colsf, rowsf, valsf):
    mesh = plsc.VectorSubcoreMesh(core_axis_name="c", subcore_axis_name="s")

    def body(y2_ref, colsf_ref, rowsf_ref, valsf_ref, out_ref, *scratch):
        cbufs = scratch[0:_P]
        rbufs = scratch[_P:2 * _P]
        vbufs = scratch[2 * _P:3 * _P]
        gbufs = scratch[3 * _P:4 * _P]
        acc = scratch[4 * _P]
        sems = scratch[4 * _P + 1:]
        _spmm_body(y2_ref, colsf_ref, rowsf_ref, valsf_ref, out_ref,
                   cbufs, rbufs, vbufs, gbufs, acc,
                   sems[0:_P], sems[_P:2 * _P], sems[2 * _P:3 * _P])

    return pl.kernel(
        body,
        out_type=jax.ShapeDtypeStruct((_NC, _NPAD, _D), jnp.float32),
        mesh=mesh,
        scratch_types=(
            [pltpu.VMEM((_C,), jnp.int32)] * _P
            + [pltpu.VMEM((_C,), jnp.int32)] * _P
            + [pltpu.VMEM((_C,), jnp.float32)] * _P
            + [pltpu.VMEM((_C, _D), jnp.float32)] * _P
            + [pltpu.VMEM_SHARED((_NPAD, _D), jnp.float32)]
            + [pltpu.SemaphoreType.DMA] * (3 * _P)
        ),
    )(y2, colsf, rowsf, valsf)


def _combine_body(p_ref, b_ref, o_ref):
    o_ref[...] = p_ref[0] + p_ref[1] + b_ref[...]


def _combine(partials, bias):
    return pl.pallas_call(
        _combine_body,
        grid=(_N // _BLK,),
        in_specs=[pl.BlockSpec((2, _BLK, _D), lambda i: (0, i, 0)),
                  pl.BlockSpec((1, _D), lambda i: (0, 0))],
        out_specs=pl.BlockSpec((_BLK, _D), lambda i: (i, 0)),
        out_shape=jax.ShapeDtypeStruct((_N, _D), jnp.float32),
    )(partials, bias)


def kernel(features, laplacian_indices, laplacian_values, selfloop_indices,
           selfloop_values, ui_indices, ui_values,
           W_lin, b_lin, W_lin1, b_lin1, W_iat, b_iat, W_iat1, b_iat1):
    y = _dense(features, W_lin, W_iat, W_lin1, W_iat1)
    y2 = y.reshape(2 * _N, _D)

    pad = _EP + 2 * _C - _E2
    cols = jnp.concatenate([
        laplacian_indices[1], ui_indices[1] + _N,
        jnp.zeros((pad,), jnp.int32)])
    rows = jnp.concatenate([
        laplacian_indices[0], ui_indices[0],
        jnp.zeros((pad,), jnp.int32)])
    vals = jnp.concatenate([
        laplacian_values, ui_values, jnp.zeros((pad,), jnp.float32)])
    partials = _spmm(y2, cols, rows, vals)

    bias = (b_lin + b_iat + b_lin1 + b_iat1).reshape(1, _D)
    return _combine(partials, bias)


# speedup vs baseline: 1.2586x; 1.0379x over previous
import jax
import jax.numpy as jnp
from jax import lax
from jax.experimental import pallas as pl
from jax.experimental.pallas import tpu as pltpu
from jax.experimental.pallas import tpu_sc as plsc

_N = 10000
_E = 320000
_D = 128

_NC = 2
_NS = 16
_NW = _NC * _NS
_L = 16

_C = 80
_P = 4
_E2 = 2 * _E
_CHUNKS = -(-_E2 // (_NW * _C * _P)) * _P
_CA = 332
_CB = 2 * _CHUNKS - _CA
_PER_TILE = _CHUNKS * _C
_EP = 2 * _PER_TILE * _NS

_NPAD = 10240
_RPS = _NPAD // _NS

_BLK = 1000


def _dense_body(f_ref, wl_ref, wi_ref, wl1_ref, wi1_ref, y_ref):
    x = f_ref[...]
    x2 = x * x
    dn = (((1,), (1,)), ((), ()))
    y_ref[0] = (lax.dot_general(x, wl_ref[...], dn, preferred_element_type=jnp.float32)
                + lax.dot_general(x2, wi_ref[...], dn, preferred_element_type=jnp.float32))
    y_ref[1] = (lax.dot_general(x, wl1_ref[...], dn, preferred_element_type=jnp.float32)
                + lax.dot_general(x2, wi1_ref[...], dn, preferred_element_type=jnp.float32))


def _dense(features, wl, wi, wl1, wi1):
    w_spec = pl.BlockSpec((_D, _D), lambda i: (0, 0))
    return pl.pallas_call(
        _dense_body,
        grid=(_N // _BLK,),
        in_specs=[pl.BlockSpec((_BLK, _D), lambda i: (i, 0))] + [w_spec] * 4,
        out_specs=pl.BlockSpec((2, _BLK, _D), lambda i: (0, i, 0)),
        out_shape=jax.ShapeDtypeStruct((2, _N, _D), jnp.float32),
    )(features, wl, wi, wl1, wi1)


def _spmm_body(y2, colsf, rowsf, valsf, out, cbufs, rbufs, vbufs, gbufs, acc, sis, sgs, sss):
    c = lax.axis_index("c")
    s = lax.axis_index("s")

    def zrow(i, carry):
        for k in range(_D // _L):
            gbufs[0][i, pl.ds(k * _L, _L)] = jnp.zeros((_L,), jnp.float32)
        return carry

    lax.fori_loop(0, _C, zrow, 0)
    for j in range(_RPS // _C):
        pltpu.sync_copy(gbufs[0], acc.at[pl.ds(s * _RPS + j * _C, _C)])
    plsc.subcore_barrier()

    cbase = s * (_CA + _CB) + c * _CA
    nchunks = jnp.where(c == 0, _CA, _CB)

    def start_idx(g, b):
        e0 = (cbase + g) * _C
        pltpu.async_copy(colsf.at[pl.ds(e0, _C)], cbufs[b], sis[b])
        pltpu.async_copy(rowsf.at[pl.ds(e0, _C)], rbufs[b], sis[b])
        pltpu.async_copy(valsf.at[pl.ds(e0, _C)], vbufs[b], sis[b])

    def wait_idx(g, b):
        e0 = (cbase + g) * _C
        pltpu.make_async_copy(colsf.at[pl.ds(e0, _C)], cbufs[b], sis[b]).wait()
        pltpu.make_async_copy(rowsf.at[pl.ds(e0, _C)], rbufs[b], sis[b]).wait()
        pltpu.make_async_copy(valsf.at[pl.ds(e0, _C)], vbufs[b], sis[b]).wait()

    def start_gather(b):
        pltpu.async_copy(y2.at[cbufs[b]], gbufs[b], sgs[b])

    def wait_gather(b):
        pltpu.make_async_copy(y2.at[cbufs[b]], gbufs[b], sgs[b]).wait()

    def start_scatter(b):
        pltpu.async_copy(gbufs[b], acc.at[rbufs[b]], sss[b], add=True)

    def wait_scatter(b):
        pltpu.make_async_copy(gbufs[b], acc.at[rbufs[b]], sss[b]).wait()

    start_idx(0, 0)
    start_idx(1, 1)
    wait_idx(0, 0)
    start_gather(0)

    def outer(g4, carry):
        for u in range(_P):
            b = u
            g = g4 * _P + u
            wait_gather(b)
            b2 = (u + 2) % _P

            @pl.when(g >= 2)
            def _():
                wait_scatter(b2)

            start_idx(g + 2, b2)
            b1 = (u + 1) % _P
            wait_idx(g + 1, b1)
            start_gather(b1)

            gb = gbufs[b]
            vb = vbufs[b]

            def grp(t, carry2):
                vvec = vb[pl.ds(t * _L, _L)]
                for j in range(_L):
                    e = t * _L + j
                    v = jnp.full((_L,), vvec[j], jnp.float32)
                    for k in range(_D // _L):
                        gb[e, pl.ds(k * _L, _L)] = gb[e, pl.ds(k * _L, _L)] * v
                return carry2

            lax.fori_loop(0, _C // _L, grp, 0)
            start_scatter(b)
        return carry

    lax.fori_loop(0, nchunks // _P, outer, 0)
    wait_gather(0)
    wait_idx(nchunks + 1, 1)
    for b in (_P - 2, _P - 1):
        wait_scatter(b)
    plsc.subcore_barrier()

    for j in range(_RPS // _C):
        r0 = s * _RPS + j * _C
        pltpu.sync_copy(acc.at[pl.ds(r0, _C)], gbufs[0])
        pltpu.sync_copy(gbufs[0], out.at[c, pl.ds(r0, _C)])


def _spmm(y2, colsf, rowsf, valsf):
    mesh = plsc.VectorSubcoreMesh(core_axis_name="c", subcore_axis_name="s")

    def body(y2_ref, colsf_ref, rowsf_ref, valsf_ref, out_ref, *scratch):
        cbufs = scratch[0:_P]
        rbufs = scratch[_P:2 * _P]
        vbufs = scratch[2 * _P:3 * _P]
        gbufs = scratch[3 * _P:4 * _P]
        acc = scratch[4 * _P]
        sems = scratch[4 * _P + 1:]
        _spmm_body(y2_ref, colsf_ref, rowsf_ref, valsf_ref, out_ref,
                   cbufs, rbufs, vbufs, gbufs, acc,
                   sems[0:_P], sems[_P:2 * _P], sems[2 * _P:3 * _P])

    return pl.kernel(
        body,
        out_type=jax.ShapeDtypeStruct((_NC, _NPAD, _D), jnp.float32),
        mesh=mesh,
        scratch_types=(
            [pltpu.VMEM((_C,), jnp.int32)] * _P
            + [pltpu.VMEM((_C,), jnp.int32)] * _P
            + [pltpu.VMEM((_C,), jnp.float32)] * _P
            + [pltpu.VMEM((_C, _D), jnp.float32)] * _P
            + [pltpu.VMEM_SHARED((_NPAD, _D), jnp.float32)]
            + [pltpu.SemaphoreType.DMA] * (3 * _P)
        ),
    )(y2, colsf, rowsf, valsf)


def _combine_body(p_ref, b_ref, o_ref):
    o_ref[...] = p_ref[0] + p_ref[1] + b_ref[...]


def _combine(partials, bias):
    return pl.pallas_call(
        _combine_body,
        grid=(_N // _BLK,),
        in_specs=[pl.BlockSpec((2, _BLK, _D), lambda i: (0, i, 0)),
                  pl.BlockSpec((1, _D), lambda i: (0, 0))],
        out_specs=pl.BlockSpec((_BLK, _D), lambda i: (i, 0)),
        out_shape=jax.ShapeDtypeStruct((_N, _D), jnp.float32),
    )(partials, bias)


def kernel(features, laplacian_indices, laplacian_values, selfloop_indices,
           selfloop_values, ui_indices, ui_values,
           W_lin, b_lin, W_lin1, b_lin1, W_iat, b_iat, W_iat1, b_iat1):
    y = _dense(features, W_lin, W_iat, W_lin1, W_iat1)
    y2 = y.reshape(2 * _N, _D)

    pad = _EP + 2 * _C - _E2
    cols = jnp.concatenate([
        laplacian_indices[1], ui_indices[1] + _N,
        jnp.zeros((pad,), jnp.int32)])
    rows = jnp.concatenate([
        laplacian_indices[0], ui_indices[0],
        jnp.zeros((pad,), jnp.int32)])
    vals = jnp.concatenate([
        laplacian_values, ui_values, jnp.zeros((pad,), jnp.float32)])
    partials = _spmm(y2, cols, rows, vals)

    bias = (b_lin + b_iat + b_lin1 + b_iat1).reshape(1, _D)
    return _combine(partials, bias)


# speedup vs baseline: 1.3652x; 1.0847x over previous
import jax
import jax.numpy as jnp
from jax import lax
from jax.experimental import pallas as pl
from jax.experimental.pallas import tpu as pltpu
from jax.experimental.pallas import tpu_sc as plsc

_N = 10000
_E = 320000
_D = 128

_NC = 2
_NS = 16
_NW = _NC * _NS
_L = 16

_C = 80
_P = 4
_E2 = 2 * _E
_CHUNKS = -(-_E2 // (_NW * _C * _P)) * _P
_CA = 332
_CB = 2 * _CHUNKS - _CA
_PER_TILE = _CHUNKS * _C
_EP = 2 * _PER_TILE * _NS

_NPAD = 10240
_RPS = _NPAD // _NS

_BLK = 1000


def _dense_body(f_ref, wl_ref, wi_ref, wl1_ref, wi1_ref, y_ref):
    x = f_ref[...]
    x2 = x * x
    dn = (((1,), (1,)), ((), ()))
    y_ref[0] = (lax.dot_general(x, wl_ref[...], dn, preferred_element_type=jnp.float32)
                + lax.dot_general(x2, wi_ref[...], dn, preferred_element_type=jnp.float32))
    y_ref[1] = (lax.dot_general(x, wl1_ref[...], dn, preferred_element_type=jnp.float32)
                + lax.dot_general(x2, wi1_ref[...], dn, preferred_element_type=jnp.float32))


def _dense(features, wl, wi, wl1, wi1):
    w_spec = pl.BlockSpec((_D, _D), lambda i: (0, 0))
    return pl.pallas_call(
        _dense_body,
        grid=(_N // _BLK,),
        in_specs=[pl.BlockSpec((_BLK, _D), lambda i: (i, 0))] + [w_spec] * 4,
        out_specs=pl.BlockSpec((2, _BLK, _D), lambda i: (0, i, 0)),
        out_shape=jax.ShapeDtypeStruct((2, _N, _D), jnp.float32),
    )(features, wl, wi, wl1, wi1)


def _spmm_body(y2, colsf, rowsf, valsf, out, cbufs, rbufs, vbufs, gbufs, acc, sis, sgs, sss):
    c = lax.axis_index("c")
    s = lax.axis_index("s")

    def zrow(i, carry):
        for k in range(_D // _L):
            gbufs[0][i, pl.ds(k * _L, _L)] = jnp.zeros((_L,), jnp.float32)
        return carry

    lax.fori_loop(0, _C, zrow, 0)
    for j in range(_RPS // _C):
        pltpu.sync_copy(gbufs[0], acc.at[pl.ds(s * _RPS + j * _C, _C)])
    plsc.subcore_barrier()

    cbase = s * (_CA + _CB) + c * _CA
    nchunks = jnp.where(c == 0, _CA, _CB)

    def start_idx(g, b):
        e0 = (cbase + g) * _C
        pltpu.async_copy(colsf.at[pl.ds(e0, _C)], cbufs[b], sis[b])
        pltpu.async_copy(rowsf.at[pl.ds(e0, _C)], rbufs[b], sis[b])
        pltpu.async_copy(valsf.at[pl.ds(e0, _C)], vbufs[b], sis[b])

    def wait_idx(g, b):
        e0 = (cbase + g) * _C
        pltpu.make_async_copy(colsf.at[pl.ds(e0, _C)], cbufs[b], sis[b]).wait()
        pltpu.make_async_copy(rowsf.at[pl.ds(e0, _C)], rbufs[b], sis[b]).wait()
        pltpu.make_async_copy(valsf.at[pl.ds(e0, _C)], vbufs[b], sis[b]).wait()

    def start_gather(b):
        pltpu.async_copy(y2.at[cbufs[b]], gbufs[b], sgs[b])

    def wait_gather(b):
        pltpu.make_async_copy(y2.at[cbufs[b]], gbufs[b], sgs[b]).wait()

    def start_scatter(b):
        pltpu.async_copy(gbufs[b], acc.at[rbufs[b]], sss[b], add=True)

    def wait_scatter(b):
        pltpu.make_async_copy(gbufs[b], acc.at[rbufs[b]], sss[b]).wait()

    start_idx(0, 0)
    start_idx(1, 1)
    wait_idx(0, 0)
    start_gather(0)

    def outer(g4, carry):
        for u in range(_P):
            b = u
            g = g4 * _P + u
            b1 = (u + 1) % _P
            wait_idx(g + 1, b1)
            start_gather(b1)
            wait_gather(b)
            b2 = (u + 2) % _P

            @pl.when(g >= 2)
            def _():
                wait_scatter(b2)

            start_idx(g + 2, b2)

            gb = gbufs[b]
            vb = vbufs[b]

            def grp(t, carry2):
                vvec = vb[pl.ds(t * _L, _L)]
                for j in range(_L):
                    e = t * _L + j
                    v = jnp.full((_L,), vvec[j], jnp.float32)
                    for k in range(_D // _L):
                        gb[e, pl.ds(k * _L, _L)] = gb[e, pl.ds(k * _L, _L)] * v
                return carry2

            lax.fori_loop(0, _C // _L, grp, 0)
            start_scatter(b)
        return carry

    lax.fori_loop(0, nchunks // _P, outer, 0)
    wait_gather(0)
    wait_idx(nchunks + 1, 1)
    for b in (_P - 2, _P - 1):
        wait_scatter(b)
    plsc.subcore_barrier()

    for j in range(_RPS // _C):
        r0 = s * _RPS + j * _C
        pltpu.sync_copy(acc.at[pl.ds(r0, _C)], gbufs[0])
        pltpu.sync_copy(gbufs[0], out.at[c, pl.ds(r0, _C)])


def _spmm(y2, colsf, rowsf, valsf):
    mesh = plsc.VectorSubcoreMesh(core_axis_name="c", subcore_axis_name="s")

    def body(y2_ref, colsf_ref, rowsf_ref, valsf_ref, out_ref, *scratch):
        cbufs = scratch[0:_P]
        rbufs = scratch[_P:2 * _P]
        vbufs = scratch[2 * _P:3 * _P]
        gbufs = scratch[3 * _P:4 * _P]
        acc = scratch[4 * _P]
        sems = scratch[4 * _P + 1:]
        _spmm_body(y2_ref, colsf_ref, rowsf_ref, valsf_ref, out_ref,
                   cbufs, rbufs, vbufs, gbufs, acc,
                   sems[0:_P], sems[_P:2 * _P], sems[2 * _P:3 * _P])

    return pl.kernel(
        body,
        out_type=jax.ShapeDtypeStruct((_NC, _NPAD, _D), jnp.float32),
        mesh=mesh,
        scratch_types=(
            [pltpu.VMEM((_C,), jnp.int32)] * _P
            + [pltpu.VMEM((_C,), jnp.int32)] * _P
            + [pltpu.VMEM((_C,), jnp.float32)] * _P
            + [pltpu.VMEM((_C, _D), jnp.float32)] * _P
            + [pltpu.VMEM_SHARED((_NPAD, _D), jnp.float32)]
            + [pltpu.SemaphoreType.DMA] * (3 * _P)
        ),
    )(y2, colsf, rowsf, valsf)


def _combine_body(p_ref, b_ref, o_ref):
    o_ref[...] = p_ref[0] + p_ref[1] + b_ref[...]


def _combine(partials, bias):
    return pl.pallas_call(
        _combine_body,
        grid=(_N // _BLK,),
        in_specs=[pl.BlockSpec((2, _BLK, _D), lambda i: (0, i, 0)),
                  pl.BlockSpec((1, _D), lambda i: (0, 0))],
        out_specs=pl.BlockSpec((_BLK, _D), lambda i: (i, 0)),
        out_shape=jax.ShapeDtypeStruct((_N, _D), jnp.float32),
    )(partials, bias)


def kernel(features, laplacian_indices, laplacian_values, selfloop_indices,
           selfloop_values, ui_indices, ui_values,
           W_lin, b_lin, W_lin1, b_lin1, W_iat, b_iat, W_iat1, b_iat1):
    y = _dense(features, W_lin, W_iat, W_lin1, W_iat1)
    y2 = y.reshape(2 * _N, _D)

    pad = _EP + 2 * _C - _E2
    cols = jnp.concatenate([
        laplacian_indices[1], ui_indices[1] + _N,
        jnp.zeros((pad,), jnp.int32)])
    rows = jnp.concatenate([
        laplacian_indices[0], ui_indices[0],
        jnp.zeros((pad,), jnp.int32)])
    vals = jnp.concatenate([
        laplacian_values, ui_values, jnp.zeros((pad,), jnp.float32)])
    partials = _spmm(y2, cols, rows, vals)

    bias = (b_lin + b_iat + b_lin1 + b_iat1).reshape(1, _D)
    return _combine(partials, bias)


# speedup vs baseline: 1.4445x; 1.0581x over previous
import jax
import jax.numpy as jnp
from jax import lax
from jax.experimental import pallas as pl
from jax.experimental.pallas import tpu as pltpu
from jax.experimental.pallas import tpu_sc as plsc

_N = 10000
_E = 320000
_D = 128

_NC = 2
_NS = 16
_NW = _NC * _NS
_L = 16

_C = 80
_P = 4
_E2 = 2 * _E
_CHUNKS = -(-_E2 // (_NW * _C * _P)) * _P
_CA = 364
_CB = 2 * _CHUNKS - _CA
_PER_TILE = _CHUNKS * _C
_EP = 2 * _PER_TILE * _NS

_NPAD = 10240
_RPS = _NPAD // _NS

_BLK = 1000


def _dense_body(f_ref, wl_ref, wi_ref, wl1_ref, wi1_ref, y_ref):
    x = f_ref[...]
    x2 = x * x
    dn = (((1,), (1,)), ((), ()))
    y_ref[0] = (lax.dot_general(x, wl_ref[...], dn, preferred_element_type=jnp.float32)
                + lax.dot_general(x2, wi_ref[...], dn, preferred_element_type=jnp.float32))
    y_ref[1] = (lax.dot_general(x, wl1_ref[...], dn, preferred_element_type=jnp.float32)
                + lax.dot_general(x2, wi1_ref[...], dn, preferred_element_type=jnp.float32))


def _dense(features, wl, wi, wl1, wi1):
    w_spec = pl.BlockSpec((_D, _D), lambda i: (0, 0))
    return pl.pallas_call(
        _dense_body,
        grid=(_N // _BLK,),
        in_specs=[pl.BlockSpec((_BLK, _D), lambda i: (i, 0))] + [w_spec] * 4,
        out_specs=pl.BlockSpec((2, _BLK, _D), lambda i: (0, i, 0)),
        out_shape=jax.ShapeDtypeStruct((2, _N, _D), jnp.float32),
    )(features, wl, wi, wl1, wi1)


def _spmm_body(y2, colsf, rowsf, valsf, out, cbufs, rbufs, vbufs, gbufs, acc, sis, sgs, sss):
    c = lax.axis_index("c")
    s = lax.axis_index("s")

    def zrow(i, carry):
        for k in range(_D // _L):
            gbufs[0][i, pl.ds(k * _L, _L)] = jnp.zeros((_L,), jnp.float32)
        return carry

    lax.fori_loop(0, _C, zrow, 0)
    for j in range(_RPS // _C):
        pltpu.sync_copy(gbufs[0], acc.at[pl.ds(s * _RPS + j * _C, _C)])
    plsc.subcore_barrier()

    cbase = s * (_CA + _CB) + c * _CA
    nchunks = jnp.where(c == 0, _CA, _CB)

    def start_idx(g, b):
        e0 = (cbase + g) * _C
        pltpu.async_copy(colsf.at[pl.ds(e0, _C)], cbufs[b], sis[b])
        pltpu.async_copy(rowsf.at[pl.ds(e0, _C)], rbufs[b], sis[b])
        pltpu.async_copy(valsf.at[pl.ds(e0, _C)], vbufs[b], sis[b])

    def wait_idx(g, b):
        e0 = (cbase + g) * _C
        pltpu.make_async_copy(colsf.at[pl.ds(e0, _C)], cbufs[b], sis[b]).wait()
        pltpu.make_async_copy(rowsf.at[pl.ds(e0, _C)], rbufs[b], sis[b]).wait()
        pltpu.make_async_copy(valsf.at[pl.ds(e0, _C)], vbufs[b], sis[b]).wait()

    def start_gather(b):
        pltpu.async_copy(y2.at[cbufs[b]], gbufs[b], sgs[b])

    def wait_gather(b):
        pltpu.make_async_copy(y2.at[cbufs[b]], gbufs[b], sgs[b]).wait()

    def start_scatter(b):
        pltpu.async_copy(gbufs[b], acc.at[rbufs[b]], sss[b], add=True)

    def wait_scatter(b):
        pltpu.make_async_copy(gbufs[b], acc.at[rbufs[b]], sss[b]).wait()

    start_idx(0, 0)
    start_idx(1, 1)
    wait_idx(0, 0)
    start_gather(0)

    def outer(g4, carry):
        for u in range(_P):
            b = u
            g = g4 * _P + u
            b1 = (u + 1) % _P
            wait_idx(g + 1, b1)
            start_gather(b1)
            wait_gather(b)
            b2 = (u + 2) % _P

            @pl.when(g >= 2)
            def _():
                wait_scatter(b2)

            start_idx(g + 2, b2)

            gb = gbufs[b]
            vb = vbufs[b]

            def grp(t, carry2):
                vvec = vb[pl.ds(t * _L, _L)]
                for j in range(_L):
                    e = t * _L + j
                    v = jnp.full((_L,), vvec[j], jnp.float32)
                    for k in range(_D // _L):
                        gb[e, pl.ds(k * _L, _L)] = gb[e, pl.ds(k * _L, _L)] * v
                return carry2

            lax.fori_loop(0, _C // _L, grp, 0)
            start_scatter(b)
        return carry

    lax.fori_loop(0, nchunks // _P, outer, 0)
    wait_gather(0)
    wait_idx(nchunks + 1, 1)
    for b in (_P - 2, _P - 1):
        wait_scatter(b)
    plsc.subcore_barrier()

    for j in range(_RPS // _C):
        r0 = s * _RPS + j * _C
        pltpu.sync_copy(acc.at[pl.ds(r0, _C)], gbufs[0])
        pltpu.sync_copy(gbufs[0], out.at[c, pl.ds(r0, _C)])


def _spmm(y2, colsf, rowsf, valsf):
    mesh = plsc.VectorSubcoreMesh(core_axis_name="c", subcore_axis_name="s")

    def body(y2_ref, colsf_ref, rowsf_ref, valsf_ref, out_ref, *scratch):
        cbufs = scratch[0:_P]
        rbufs = scratch[_P:2 * _P]
        vbufs = scratch[2 * _P:3 * _P]
        gbufs = scratch[3 * _P:4 * _P]
        acc = scratch[4 * _P]
        sems = scratch[4 * _P + 1:]
        _spmm_body(y2_ref, colsf_ref, rowsf_ref, valsf_ref, out_ref,
                   cbufs, rbufs, vbufs, gbufs, acc,
                   sems[0:_P], sems[_P:2 * _P], sems[2 * _P:3 * _P])

    return pl.kernel(
        body,
        out_type=jax.ShapeDtypeStruct((_NC, _NPAD, _D), jnp.float32),
        mesh=mesh,
        scratch_types=(
            [pltpu.VMEM((_C,), jnp.int32)] * _P
            + [pltpu.VMEM((_C,), jnp.int32)] * _P
            + [pltpu.VMEM((_C,), jnp.float32)] * _P
            + [pltpu.VMEM((_C, _D), jnp.float32)] * _P
            + [pltpu.VMEM_SHARED((_NPAD, _D), jnp.float32)]
            + [pltpu.SemaphoreType.DMA] * (3 * _P)
        ),
    )(y2, colsf, rowsf, valsf)


def _combine_body(p_ref, b_ref, o_ref):
    o_ref[...] = p_ref[0] + p_ref[1] + b_ref[...]


def _combine(partials, bias):
    return pl.pallas_call(
        _combine_body,
        grid=(_N // _BLK,),
        in_specs=[pl.BlockSpec((2, _BLK, _D), lambda i: (0, i, 0)),
                  pl.BlockSpec((1, _D), lambda i: (0, 0))],
        out_specs=pl.BlockSpec((_BLK, _D), lambda i: (i, 0)),
        out_shape=jax.ShapeDtypeStruct((_N, _D), jnp.float32),
    )(partials, bias)


def kernel(features, laplacian_indices, laplacian_values, selfloop_indices,
           selfloop_values, ui_indices, ui_values,
           W_lin, b_lin, W_lin1, b_lin1, W_iat, b_iat, W_iat1, b_iat1):
    y = _dense(features, W_lin, W_iat, W_lin1, W_iat1)
    y2 = y.reshape(2 * _N, _D)

    pad = _EP + 2 * _C - _E2
    cols = jnp.concatenate([
        laplacian_indices[1], ui_indices[1] + _N,
        jnp.zeros((pad,), jnp.int32)])
    rows = jnp.concatenate([
        laplacian_indices[0], ui_indices[0],
        jnp.zeros((pad,), jnp.int32)])
    vals = jnp.concatenate([
        laplacian_values, ui_values, jnp.zeros((pad,), jnp.float32)])
    partials = _spmm(y2, cols, rows, vals)

    bias = (b_lin + b_iat + b_lin1 + b_iat1).reshape(1, _D)
    return _combine(partials, bias)


# speedup vs baseline: 1.5326x; 1.0610x over previous
import jax
import jax.numpy as jnp
from jax import lax
from jax.experimental import pallas as pl
from jax.experimental.pallas import tpu as pltpu
from jax.experimental.pallas import tpu_sc as plsc

_N = 10000
_E = 320000
_D = 128

_NC = 2
_NS = 16
_NW = _NC * _NS
_L = 16

_C = 80
_P = 4
_E2 = 2 * _E
_CHUNKS = -(-_E2 // (_NW * _C * _P)) * _P
_CA = 396
_CB = 2 * _CHUNKS - _CA
_PER_TILE = _CHUNKS * _C
_EP = 2 * _PER_TILE * _NS

_NPAD = 10240
_RPS = _NPAD // _NS

_BLK = 1000


def _dense_body(f_ref, wl_ref, wi_ref, wl1_ref, wi1_ref, y_ref):
    x = f_ref[...]
    x2 = x * x
    dn = (((1,), (1,)), ((), ()))
    y_ref[0] = (lax.dot_general(x, wl_ref[...], dn, preferred_element_type=jnp.float32)
                + lax.dot_general(x2, wi_ref[...], dn, preferred_element_type=jnp.float32))
    y_ref[1] = (lax.dot_general(x, wl1_ref[...], dn, preferred_element_type=jnp.float32)
                + lax.dot_general(x2, wi1_ref[...], dn, preferred_element_type=jnp.float32))


def _dense(features, wl, wi, wl1, wi1):
    w_spec = pl.BlockSpec((_D, _D), lambda i: (0, 0))
    return pl.pallas_call(
        _dense_body,
        grid=(_N // _BLK,),
        in_specs=[pl.BlockSpec((_BLK, _D), lambda i: (i, 0))] + [w_spec] * 4,
        out_specs=pl.BlockSpec((2, _BLK, _D), lambda i: (0, i, 0)),
        out_shape=jax.ShapeDtypeStruct((2, _N, _D), jnp.float32),
    )(features, wl, wi, wl1, wi1)


def _spmm_body(y2, colsf, rowsf, valsf, out, cbufs, rbufs, vbufs, gbufs, acc, sis, sgs, sss):
    c = lax.axis_index("c")
    s = lax.axis_index("s")

    def zrow(i, carry):
        for k in range(_D // _L):
            gbufs[0][i, pl.ds(k * _L, _L)] = jnp.zeros((_L,), jnp.float32)
        return carry

    lax.fori_loop(0, _C, zrow, 0)
    for j in range(_RPS // _C):
        pltpu.sync_copy(gbufs[0], acc.at[pl.ds(s * _RPS + j * _C, _C)])
    plsc.subcore_barrier()

    cbase = s * (_CA + _CB) + c * _CA
    nchunks = jnp.where(c == 0, _CA, _CB)

    def start_idx(g, b):
        e0 = (cbase + g) * _C
        pltpu.async_copy(colsf.at[pl.ds(e0, _C)], cbufs[b], sis[b])
        pltpu.async_copy(rowsf.at[pl.ds(e0, _C)], rbufs[b], sis[b])
        pltpu.async_copy(valsf.at[pl.ds(e0, _C)], vbufs[b], sis[b])

    def wait_idx(g, b):
        e0 = (cbase + g) * _C
        pltpu.make_async_copy(colsf.at[pl.ds(e0, _C)], cbufs[b], sis[b]).wait()
        pltpu.make_async_copy(rowsf.at[pl.ds(e0, _C)], rbufs[b], sis[b]).wait()
        pltpu.make_async_copy(valsf.at[pl.ds(e0, _C)], vbufs[b], sis[b]).wait()

    def start_gather(b):
        pltpu.async_copy(y2.at[cbufs[b]], gbufs[b], sgs[b])

    def wait_gather(b):
        pltpu.make_async_copy(y2.at[cbufs[b]], gbufs[b], sgs[b]).wait()

    def start_scatter(b):
        pltpu.async_copy(gbufs[b], acc.at[rbufs[b]], sss[b], add=True)

    def wait_scatter(b):
        pltpu.make_async_copy(gbufs[b], acc.at[rbufs[b]], sss[b]).wait()

    start_idx(0, 0)
    start_idx(1, 1)
    wait_idx(0, 0)
    start_gather(0)

    def outer(g4, carry):
        for u in range(_P):
            b = u
            g = g4 * _P + u
            b1 = (u + 1) % _P
            wait_idx(g + 1, b1)
            start_gather(b1)
            wait_gather(b)
            b2 = (u + 2) % _P

            @pl.when(g >= 2)
            def _():
                wait_scatter(b2)

            start_idx(g + 2, b2)

            gb = gbufs[b]
            vb = vbufs[b]

            def grp(t, carry2):
                vvec = vb[pl.ds(t * _L, _L)]
                for j in range(_L):
                    e = t * _L + j
                    v = jnp.full((_L,), vvec[j], jnp.float32)
                    for k in range(_D // _L):
                        gb[e, pl.ds(k * _L, _L)] = gb[e, pl.ds(k * _L, _L)] * v
                return carry2

            lax.fori_loop(0, _C // _L, grp, 0)
            start_scatter(b)
        return carry

    lax.fori_loop(0, nchunks // _P, outer, 0)
    wait_gather(0)
    wait_idx(nchunks + 1, 1)
    for b in (_P - 2, _P - 1):
        wait_scatter(b)
    plsc.subcore_barrier()

    for j in range(_RPS // _C):
        r0 = s * _RPS + j * _C
        pltpu.sync_copy(acc.at[pl.ds(r0, _C)], gbufs[0])
        pltpu.sync_copy(gbufs[0], out.at[c, pl.ds(r0, _C)])


def _spmm(y2, colsf, rowsf, valsf):
    mesh = plsc.VectorSubcoreMesh(core_axis_name="c", subcore_axis_name="s")

    def body(y2_ref, colsf_ref, rowsf_ref, valsf_ref, out_ref, *scratch):
        cbufs = scratch[0:_P]
        rbufs = scratch[_P:2 * _P]
        vbufs = scratch[2 * _P:3 * _P]
        gbufs = scratch[3 * _P:4 * _P]
        acc = scratch[4 * _P]
        sems = scratch[4 * _P + 1:]
        _spmm_body(y2_ref, colsf_ref, rowsf_ref, valsf_ref, out_ref,
                   cbufs, rbufs, vbufs, gbufs, acc,
                   sems[0:_P], sems[_P:2 * _P], sems[2 * _P:3 * _P])

    return pl.kernel(
        body,
        out_type=jax.ShapeDtypeStruct((_NC, _NPAD, _D), jnp.float32),
        mesh=mesh,
        scratch_types=(
            [pltpu.VMEM((_C,), jnp.int32)] * _P
            + [pltpu.VMEM((_C,), jnp.int32)] * _P
            + [pltpu.VMEM((_C,), jnp.float32)] * _P
            + [pltpu.VMEM((_C, _D), jnp.float32)] * _P
            + [pltpu.VMEM_SHARED((_NPAD, _D), jnp.float32)]
            + [pltpu.SemaphoreType.DMA] * (3 * _P)
        ),
    )(y2, colsf, rowsf, valsf)


def _combine_body(p_ref, b_ref, o_ref):
    o_ref[...] = p_ref[0] + p_ref[1] + b_ref[...]


def _combine(partials, bias):
    return pl.pallas_call(
        _combine_body,
        grid=(_N // _BLK,),
        in_specs=[pl.BlockSpec((2, _BLK, _D), lambda i: (0, i, 0)),
                  pl.BlockSpec((1, _D), lambda i: (0, 0))],
        out_specs=pl.BlockSpec((_BLK, _D), lambda i: (i, 0)),
        out_shape=jax.ShapeDtypeStruct((_N, _D), jnp.float32),
    )(partials, bias)


def kernel(features, laplacian_indices, laplacian_values, selfloop_indices,
           selfloop_values, ui_indices, ui_values,
           W_lin, b_lin, W_lin1, b_lin1, W_iat, b_iat, W_iat1, b_iat1):
    y = _dense(features, W_lin, W_iat, W_lin1, W_iat1)
    y2 = y.reshape(2 * _N, _D)

    pad = _EP + 2 * _C - _E2
    cols = jnp.concatenate([
        laplacian_indices[1], ui_indices[1] + _N,
        jnp.zeros((pad,), jnp.int32)])
    rows = jnp.concatenate([
        laplacian_indices[0], ui_indices[0],
        jnp.zeros((pad,), jnp.int32)])
    vals = jnp.concatenate([
        laplacian_values, ui_values, jnp.zeros((pad,), jnp.float32)])
    partials = _spmm(y2, cols, rows, vals)

    bias = (b_lin + b_iat + b_lin1 + b_iat1).reshape(1, _D)
    return _combine(partials, bias)
